# baseline (speedup 1.0000x reference)
import functools

import jax
import jax.numpy as jnp
from jax import lax
from jax.experimental import pallas as pl
from jax.experimental.pallas import tpu as pltpu
from jax.experimental.pallas import tpu_sc as plsc

N = 10000
E = 320000
D = 128

H = E // 2
NW = 32
EWH = H // NW
CBH = 40
NCBH = EWH // CBH

BEH = 1280
BN = 512
NP = 10240
RPT = NP // 16

_mesh = functools.partial(
    plsc.VectorSubcoreMesh, core_axis_name="c", subcore_axis_name="s")


def _make_gather_body(half):
    def body(x_hbm, src_hbm, out_hbm,
             idx0, idx1, rows0, rows1,
             i_sem0, i_sem1, g_sem0, g_sem1, w_sem0, w_sem1):
        w = lax.axis_index("c") * 16 + lax.axis_index("s")
        base = half * H + w * EWH
        obase = w * EWH
        bufs = ((idx0, rows0, i_sem0, g_sem0, w_sem0),
                (idx1, rows1, i_sem1, g_sem1, w_sem1))

        def start_idx(i, b):
            idx_v, _, i_sem, _, _ = bufs[b]
            pltpu.async_copy(src_hbm.at[pl.ds(base + i * CBH, CBH)], idx_v, i_sem)

        def drain_idx(i, b):
            idx_v, _, i_sem, _, _ = bufs[b]
            pltpu.make_async_copy(src_hbm.at[pl.ds(base + i * CBH, CBH)], idx_v,
                                  i_sem).wait()

        def fire_gather(b):
            idx_v, rows_v, _, g_sem, _ = bufs[b]
            pltpu.async_copy(x_hbm.at[idx_v], rows_v, g_sem)

        def drain_gather(b):
            idx_v, rows_v, _, g_sem, _ = bufs[b]
            pltpu.make_async_copy(x_hbm.at[idx_v], rows_v, g_sem).wait()

        def fire_write(i, b):
            _, rows_v, _, _, w_sem = bufs[b]
            pltpu.async_copy(rows_v, out_hbm.at[pl.ds(obase + i * CBH, CBH)], w_sem)

        def drain_write(i, b):
            _, rows_v, _, _, w_sem = bufs[b]
            pltpu.make_async_copy(rows_v, out_hbm.at[pl.ds(obase + i * CBH, CBH)],
                                  w_sem).wait()

        def chunk(i, b):
            drain_idx(i, b)
            pl.when(i >= 2)(lambda: drain_write(i - 2, b))
            fire_gather(b)

            def _prev():
                drain_gather(1 - b)
                fire_write(i - 1, 1 - b)
            pl.when(i >= 1)(_prev)
            pl.when(i + 1 < NCBH)(lambda: start_idx(i + 1, 1 - b))

        start_idx(0, 0)

        def pair(g, carry):
            chunk(2 * g, 0)
            chunk(2 * g + 1, 1)
            return carry

        lax.fori_loop(0, (NCBH - 1) // 2, pair, 0)
        chunk(NCBH - 1, 0)
        drain_gather(0)
        fire_write(NCBH - 1, 0)
        drain_write(NCBH - 2, 1)
        drain_write(NCBH - 1, 0)

    return body


def _make_gather(half):
    @jax.jit
    def gather(x, src):
        k = pl.kernel(
            _make_gather_body(half),
            out_type=jax.ShapeDtypeStruct((H, D), jnp.float32),
            mesh=_mesh(),
            scratch_types=[
                pltpu.VMEM((CBH,), jnp.int32),
                pltpu.VMEM((CBH,), jnp.int32),
                pltpu.VMEM((CBH, D), jnp.float32),
                pltpu.VMEM((CBH, D), jnp.float32),
                pltpu.SemaphoreType.DMA,
                pltpu.SemaphoreType.DMA,
                pltpu.SemaphoreType.DMA,
                pltpu.SemaphoreType.DMA,
                pltpu.SemaphoreType.DMA,
                pltpu.SemaphoreType.DMA,
            ],
        )
        return k(x, src)
    return gather


_gather_a = _make_gather(0)
_gather_b = _make_gather(1)


def _edge_body(xs_ref, eh_ref, eq_ref, wmx_ref, wmh_ref, mx_ref, mh_ref,
               wmsg_ref, e_ref):
    xs = xs_ref[...]
    eh = eh_ref[...]
    eq = eq_ref[...]
    xsb = xs.astype(jnp.bfloat16)
    ehb = eh.astype(jnp.bfloat16)
    eqb = eq.astype(jnp.bfloat16)
    pre = (jnp.dot(xsb, wmx_ref[...], preferred_element_type=jnp.float32)
           + jnp.dot(ehb, wmh_ref[...], preferred_element_type=jnp.float32))
    msg = jnp.where(pre > 0, pre, 0.01 * pre)
    zx = jnp.dot(eqb, mx_ref[...], preferred_element_type=jnp.float32)
    zh = jnp.dot(eqb, mh_ref[...], preferred_element_type=jnp.float32)
    att = jnp.sum(zx * xs + zh * eh, axis=1)
    e = jnp.exp(att)
    wmsg_ref[...] = e[:, None] * msg
    e_ref[...] = e[None, None, :]


def _make_edge_tc(half):
    off = half * (H // BEH)

    @jax.jit
    def edge_tc(xs, eh, eq, wmxT, wmhT, mx, mh):
        grid = H // BEH
        wspec = pl.BlockSpec((D, D), lambda i: (0, 0))
        return pl.pallas_call(
            _edge_body,
            grid=(grid,),
            in_specs=[
                pl.BlockSpec((BEH, D), lambda i: (i, 0)),
                pl.BlockSpec((BEH, D), lambda i: (i + off, 0)),
                pl.BlockSpec((BEH, D), lambda i: (i + off, 0)),
                wspec, wspec, wspec, wspec,
            ],
            out_specs=[
                pl.BlockSpec((BEH, D), lambda i: (i, 0)),
                pl.BlockSpec((1, 1, BEH), lambda i: (i, 0, 0)),
            ],
            out_shape=[
                jax.ShapeDtypeStruct((H, D), jnp.float32),
                jax.ShapeDtypeStruct((H // BEH, 1, BEH), jnp.float32),
            ],
        )(xs, eh, eq, wmxT, wmhT, mx, mh)
    return edge_tc


_edge_tc_a = _make_edge_tc(0)
_edge_tc_b = _make_edge_tc(1)


def _make_scatter_body(half):
    def body(wmsg_hbm, e_hbm, dst_hbm, zrows_hbm, zden_hbm,
             hagg_hbm, den_hbm,
             rows0, rows1, ev0, ev1, idx0, idx1,
             table, den_sp, in_sem0, in_sem1, sc_sem0, sc_sem1):
        c = lax.axis_index("c")
        s = lax.axis_index("s")
        w = c * 16 + s

        pltpu.sync_copy(zrows_hbm.at[pl.ds(s * RPT, RPT)],
                        table.at[pl.ds(s * RPT, RPT)])
        pltpu.sync_copy(zden_hbm.at[pl.ds(s * RPT, RPT)],
                        den_sp.at[pl.ds(s * RPT, RPT)])
        plsc.subcore_barrier()

        base = w * EWH
        dbase = half * H + w * EWH
        bufs = ((rows0, ev0, idx0, in_sem0, sc_sem0),
                (rows1, ev1, idx1, in_sem1, sc_sem1))

        def start_inputs(i, b):
            rows_v, ev_v, idx_v, in_sem, _ = bufs[b]
            off = base + i * CBH
            pltpu.async_copy(wmsg_hbm.at[pl.ds(off, CBH)], rows_v, in_sem)
            pltpu.async_copy(e_hbm.at[pl.ds(off, CBH)], ev_v, in_sem)
            pltpu.async_copy(dst_hbm.at[pl.ds(dbase + i * CBH, CBH)], idx_v, in_sem)

        def drain_inputs(i, b):
            rows_v, ev_v, idx_v, in_sem, _ = bufs[b]
            off = base + i * CBH
            pltpu.make_async_copy(wmsg_hbm.at[pl.ds(off, CBH)], rows_v,
                                  in_sem).wait()
            pltpu.make_async_copy(e_hbm.at[pl.ds(off, CBH)], ev_v, in_sem).wait()
            pltpu.make_async_copy(dst_hbm.at[pl.ds(dbase + i * CBH, CBH)], idx_v,
                                  in_sem).wait()

        def fire_scatters(b):
            rows_v, ev_v, idx_v, _, sc_sem = bufs[b]
            pltpu.async_copy(rows_v, table.at[idx_v], sc_sem, add=True)
            pltpu.async_copy(ev_v, den_sp.at[idx_v], sc_sem, add=True)

        def drain_scatters(b):
            rows_v, ev_v, idx_v, _, sc_sem = bufs[b]
            pltpu.make_async_copy(rows_v, table.at[idx_v], sc_sem).wait()
            pltpu.make_async_copy(ev_v, den_sp.at[idx_v], sc_sem).wait()

        start_inputs(0, 0)

        def pair(g, carry):
            i0 = 2 * g
            drain_inputs(i0, 0)
            fire_scatters(0)
            pl.when(i0 > 0)(lambda: drain_scatters(1))
            start_inputs(i0 + 1, 1)

            drain_inputs(i0 + 1, 1)
            fire_scatters(1)
            drain_scatters(0)
            start_inputs(i0 + 2, 0)
            return carry

        lax.fori_loop(0, (NCBH - 1) // 2, pair, 0)
        drain_inputs(NCBH - 1, 0)
        fire_scatters(0)
        drain_scatters(1)
        drain_scatters(0)
        plsc.subcore_barrier()

        pltpu.sync_copy(table.at[pl.ds(s * RPT, RPT)],
                        hagg_hbm.at[c, pl.ds(s * RPT, RPT)])
        pltpu.sync_copy(den_sp.at[pl.ds(s * RPT, RPT)],
                        den_hbm.at[c, pl.ds(s * RPT, RPT)])

    return body


def _make_scatter(half):
    @jax.jit
    def scatter(wmsg, e, dst):
        zrows = jnp.zeros((NP, D), jnp.float32)
        zden = jnp.zeros((NP,), jnp.float32)
        k = pl.kernel(
            _make_scatter_body(half),
            out_type=[
                jax.ShapeDtypeStruct((2, NP, D), jnp.float32),
                jax.ShapeDtypeStruct((2, NP), jnp.float32),
            ],
            mesh=_mesh(),
            scratch_types=[
                pltpu.VMEM((CBH, D), jnp.float32),
                pltpu.VMEM((CBH, D), jnp.float32),
                pltpu.VMEM((CBH,), jnp.float32),
                pltpu.VMEM((CBH,), jnp.float32),
                pltpu.VMEM((CBH,), jnp.int32),
                pltpu.VMEM((CBH,), jnp.int32),
                pltpu.VMEM_SHARED((NP, D), jnp.float32),
                pltpu.VMEM_SHARED((NP,), jnp.float32),
                pltpu.SemaphoreType.DMA,
                pltpu.SemaphoreType.DMA,
                pltpu.SemaphoreType.DMA,
                pltpu.SemaphoreType.DMA,
            ],
        )
        return k(wmsg, e, dst, zrows, zden)
    return scatter


_scatter_a = _make_scatter(0)
_scatter_b = _make_scatter(1)


def _final_body(ha_ref, da_ref, hb_ref, db_ref, x_ref, g_ref, b_ref, out_ref):
    hs = ha_ref[0] + ha_ref[1] + hb_ref[0] + hb_ref[1]
    dn = da_ref[0] + da_ref[1] + db_ref[0] + db_ref[1]
    dn = jnp.where(dn == 0.0, 1.0, dn)
    h = hs / dn[:, None] + x_ref[...]
    mean = jnp.mean(h, axis=1, keepdims=True)
    cen = h - mean
    var = jnp.mean(cen * cen, axis=1, keepdims=True)
    out_ref[...] = cen * lax.rsqrt(var + 1e-6) * g_ref[...] + b_ref[...]


@jax.jit
def _final(ha, da, hb, db, x, gamma, beta):
    grid = pl.cdiv(N, BN)
    hspec = pl.BlockSpec((2, BN, D), lambda i: (0, i, 0))
    dspec = pl.BlockSpec((2, BN), lambda i: (0, i))
    return pl.pallas_call(
        _final_body,
        grid=(grid,),
        in_specs=[
            hspec, dspec, hspec, dspec,
            pl.BlockSpec((BN, D), lambda i: (i, 0)),
            pl.BlockSpec((1, D), lambda i: (0, 0)),
            pl.BlockSpec((1, D), lambda i: (0, 0)),
        ],
        out_specs=pl.BlockSpec((BN, D), lambda i: (i, 0)),
        out_shape=jax.ShapeDtypeStruct((N, D), jnp.float32),
    )(ha, da, hb, db, x, gamma, beta)


def kernel(x, edge_index, edge_h, edge_qrh, W_msg, W_q, W_k, gamma, beta):
    src = edge_index[0].astype(jnp.int32)
    dst = edge_index[1].astype(jnp.int32)
    temp = jnp.float32(D ** 0.5)

    wmxT = W_msg[:, :D].T.astype(jnp.bfloat16)
    wmhT = W_msg[:, D:].T.astype(jnp.bfloat16)
    m = (W_q.T @ W_k) / temp
    mx = m[:, :D].astype(jnp.bfloat16)
    mh = m[:, D:].astype(jnp.bfloat16)

    xs_a = _gather_a(x, src)
    wmsg_a, e2d_a = _edge_tc_a(xs_a, edge_h, edge_qrh, wmxT, wmhT, mx, mh)
    xs_b = _gather_b(x, src)
    wmsg_b, e2d_b = _edge_tc_b(xs_b, edge_h, edge_qrh, wmxT, wmhT, mx, mh)
    ha, da = _scatter_a(wmsg_a, e2d_a.reshape(H), dst)
    hb, db = _scatter_b(wmsg_b, e2d_b.reshape(H), dst)
    return _final(ha, da, hb, db, x, gamma.reshape(1, D), beta.reshape(1, D))


# speedup vs baseline: 1.0829x; 1.0829x over previous
import functools

import jax
import jax.numpy as jnp
from jax import lax
from jax.experimental import pallas as pl
from jax.experimental.pallas import tpu as pltpu
from jax.experimental.pallas import tpu_sc as plsc

N = 10000
E = 320000
D = 128

H = E // 2
NW = 32
EWH = H // NW
CBH = 40
NCBH = EWH // CBH

BEH = 3200
BN = 512
NP = 10240
RPT = NP // 16

_mesh = functools.partial(
    plsc.VectorSubcoreMesh, core_axis_name="c", subcore_axis_name="s")


def _make_gather_body(half):
    def body(x_hbm, src_hbm, out_hbm,
             idx0, idx1, rows0, rows1,
             i_sem0, i_sem1, g_sem0, g_sem1, w_sem0, w_sem1):
        w = lax.axis_index("c") * 16 + lax.axis_index("s")
        base = half * H + w * EWH
        obase = w * EWH
        bufs = ((idx0, rows0, i_sem0, g_sem0, w_sem0),
                (idx1, rows1, i_sem1, g_sem1, w_sem1))

        def start_idx(i, b):
            idx_v, _, i_sem, _, _ = bufs[b]
            pltpu.async_copy(src_hbm.at[pl.ds(base + i * CBH, CBH)], idx_v, i_sem)

        def drain_idx(i, b):
            idx_v, _, i_sem, _, _ = bufs[b]
            pltpu.make_async_copy(src_hbm.at[pl.ds(base + i * CBH, CBH)], idx_v,
                                  i_sem).wait()

        def fire_gather(b):
            idx_v, rows_v, _, g_sem, _ = bufs[b]
            pltpu.async_copy(x_hbm.at[idx_v], rows_v, g_sem)

        def drain_gather(b):
            idx_v, rows_v, _, g_sem, _ = bufs[b]
            pltpu.make_async_copy(x_hbm.at[idx_v], rows_v, g_sem).wait()

        def fire_write(i, b):
            _, rows_v, _, _, w_sem = bufs[b]
            pltpu.async_copy(rows_v, out_hbm.at[pl.ds(obase + i * CBH, CBH)], w_sem)

        def drain_write(i, b):
            _, rows_v, _, _, w_sem = bufs[b]
            pltpu.make_async_copy(rows_v, out_hbm.at[pl.ds(obase + i * CBH, CBH)],
                                  w_sem).wait()

        def chunk(i, b):
            drain_idx(i, b)
            pl.when(i >= 2)(lambda: drain_write(i - 2, b))
            fire_gather(b)

            def _prev():
                drain_gather(1 - b)
                fire_write(i - 1, 1 - b)
            pl.when(i >= 1)(_prev)
            pl.when(i + 1 < NCBH)(lambda: start_idx(i + 1, 1 - b))

        start_idx(0, 0)

        def pair(g, carry):
            chunk(2 * g, 0)
            chunk(2 * g + 1, 1)
            return carry

        lax.fori_loop(0, (NCBH - 1) // 2, pair, 0)
        chunk(NCBH - 1, 0)
        drain_gather(0)
        fire_write(NCBH - 1, 0)
        drain_write(NCBH - 2, 1)
        drain_write(NCBH - 1, 0)

    return body


def _make_gather(half):
    @jax.jit
    def gather(x, src):
        k = pl.kernel(
            _make_gather_body(half),
            out_type=jax.ShapeDtypeStruct((H, D), jnp.float32),
            mesh=_mesh(),
            scratch_types=[
                pltpu.VMEM((CBH,), jnp.int32),
                pltpu.VMEM((CBH,), jnp.int32),
                pltpu.VMEM((CBH, D), jnp.float32),
                pltpu.VMEM((CBH, D), jnp.float32),
                pltpu.SemaphoreType.DMA,
                pltpu.SemaphoreType.DMA,
                pltpu.SemaphoreType.DMA,
                pltpu.SemaphoreType.DMA,
                pltpu.SemaphoreType.DMA,
                pltpu.SemaphoreType.DMA,
            ],
        )
        return k(x, src)
    return gather


_gather_a = _make_gather(0)
_gather_b = _make_gather(1)


def _edge_body(xs_ref, eh_ref, eq_ref, wmx_ref, wmh_ref, mx_ref, mh_ref,
               wmsg_ref, e_ref):
    xs = xs_ref[...]
    eh = eh_ref[...]
    eq = eq_ref[...]
    xsb = xs.astype(jnp.bfloat16)
    ehb = eh.astype(jnp.bfloat16)
    eqb = eq.astype(jnp.bfloat16)
    pre = (jnp.dot(xsb, wmx_ref[...], preferred_element_type=jnp.float32)
           + jnp.dot(ehb, wmh_ref[...], preferred_element_type=jnp.float32))
    msg = jnp.where(pre > 0, pre, 0.01 * pre)
    zx = jnp.dot(eqb, mx_ref[...], preferred_element_type=jnp.float32)
    zh = jnp.dot(eqb, mh_ref[...], preferred_element_type=jnp.float32)
    att = jnp.sum(zx * xs + zh * eh, axis=1)
    e = jnp.exp(att)
    wmsg_ref[...] = e[:, None] * msg
    e_ref[...] = e[None, None, :]


def _make_edge_tc(half):
    off = half * (H // BEH)

    @jax.jit
    def edge_tc(xs, eh, eq, wmxT, wmhT, mx, mh):
        grid = H // BEH
        wspec = pl.BlockSpec((D, D), lambda i: (0, 0))
        return pl.pallas_call(
            _edge_body,
            grid=(grid,),
            in_specs=[
                pl.BlockSpec((BEH, D), lambda i: (i, 0)),
                pl.BlockSpec((BEH, D), lambda i: (i + off, 0)),
                pl.BlockSpec((BEH, D), lambda i: (i + off, 0)),
                wspec, wspec, wspec, wspec,
            ],
            out_specs=[
                pl.BlockSpec((BEH, D), lambda i: (i, 0)),
                pl.BlockSpec((1, 1, BEH), lambda i: (i, 0, 0)),
            ],
            out_shape=[
                jax.ShapeDtypeStruct((H, D), jnp.float32),
                jax.ShapeDtypeStruct((H // BEH, 1, BEH), jnp.float32),
            ],
        )(xs, eh, eq, wmxT, wmhT, mx, mh)
    return edge_tc


_edge_tc_a = _make_edge_tc(0)
_edge_tc_b = _make_edge_tc(1)


def _make_scatter_body(half):
    def body(wmsg_hbm, e_hbm, dst_hbm, zrows_hbm, zden_hbm,
             hagg_hbm, den_hbm,
             rows0, rows1, ev0, ev1, idx0, idx1,
             table, den_sp, in_sem0, in_sem1, sc_sem0, sc_sem1):
        c = lax.axis_index("c")
        s = lax.axis_index("s")
        w = c * 16 + s

        pltpu.sync_copy(zrows_hbm.at[pl.ds(s * RPT, RPT)],
                        table.at[pl.ds(s * RPT, RPT)])
        pltpu.sync_copy(zden_hbm.at[pl.ds(s * RPT, RPT)],
                        den_sp.at[pl.ds(s * RPT, RPT)])
        plsc.subcore_barrier()

        base = w * EWH
        dbase = half * H + w * EWH
        bufs = ((rows0, ev0, idx0, in_sem0, sc_sem0),
                (rows1, ev1, idx1, in_sem1, sc_sem1))

        def start_inputs(i, b):
            rows_v, ev_v, idx_v, in_sem, _ = bufs[b]
            off = base + i * CBH
            pltpu.async_copy(wmsg_hbm.at[pl.ds(off, CBH)], rows_v, in_sem)
            pltpu.async_copy(e_hbm.at[pl.ds(off, CBH)], ev_v, in_sem)
            pltpu.async_copy(dst_hbm.at[pl.ds(dbase + i * CBH, CBH)], idx_v, in_sem)

        def drain_inputs(i, b):
            rows_v, ev_v, idx_v, in_sem, _ = bufs[b]
            off = base + i * CBH
            pltpu.make_async_copy(wmsg_hbm.at[pl.ds(off, CBH)], rows_v,
                                  in_sem).wait()
            pltpu.make_async_copy(e_hbm.at[pl.ds(off, CBH)], ev_v, in_sem).wait()
            pltpu.make_async_copy(dst_hbm.at[pl.ds(dbase + i * CBH, CBH)], idx_v,
                                  in_sem).wait()

        def fire_scatters(b):
            rows_v, ev_v, idx_v, _, sc_sem = bufs[b]
            pltpu.async_copy(rows_v, table.at[idx_v], sc_sem, add=True)
            pltpu.async_copy(ev_v, den_sp.at[idx_v], sc_sem, add=True)

        def drain_scatters(b):
            rows_v, ev_v, idx_v, _, sc_sem = bufs[b]
            pltpu.make_async_copy(rows_v, table.at[idx_v], sc_sem).wait()
            pltpu.make_async_copy(ev_v, den_sp.at[idx_v], sc_sem).wait()

        start_inputs(0, 0)

        def pair(g, carry):
            i0 = 2 * g
            drain_inputs(i0, 0)
            fire_scatters(0)
            pl.when(i0 > 0)(lambda: drain_scatters(1))
            start_inputs(i0 + 1, 1)

            drain_inputs(i0 + 1, 1)
            fire_scatters(1)
            drain_scatters(0)
            start_inputs(i0 + 2, 0)
            return carry

        lax.fori_loop(0, (NCBH - 1) // 2, pair, 0)
        drain_inputs(NCBH - 1, 0)
        fire_scatters(0)
        drain_scatters(1)
        drain_scatters(0)
        plsc.subcore_barrier()

        pltpu.sync_copy(table.at[pl.ds(s * RPT, RPT)],
                        hagg_hbm.at[c, pl.ds(s * RPT, RPT)])
        pltpu.sync_copy(den_sp.at[pl.ds(s * RPT, RPT)],
                        den_hbm.at[c, pl.ds(s * RPT, RPT)])

    return body


def _make_scatter(half):
    @jax.jit
    def scatter(wmsg, e, dst):
        zrows = jnp.zeros((NP, D), jnp.float32)
        zden = jnp.zeros((NP,), jnp.float32)
        k = pl.kernel(
            _make_scatter_body(half),
            out_type=[
                jax.ShapeDtypeStruct((2, NP, D), jnp.float32),
                jax.ShapeDtypeStruct((2, NP), jnp.float32),
            ],
            mesh=_mesh(),
            scratch_types=[
                pltpu.VMEM((CBH, D), jnp.float32),
                pltpu.VMEM((CBH, D), jnp.float32),
                pltpu.VMEM((CBH,), jnp.float32),
                pltpu.VMEM((CBH,), jnp.float32),
                pltpu.VMEM((CBH,), jnp.int32),
                pltpu.VMEM((CBH,), jnp.int32),
                pltpu.VMEM_SHARED((NP, D), jnp.float32),
                pltpu.VMEM_SHARED((NP,), jnp.float32),
                pltpu.SemaphoreType.DMA,
                pltpu.SemaphoreType.DMA,
                pltpu.SemaphoreType.DMA,
                pltpu.SemaphoreType.DMA,
            ],
        )
        return k(wmsg, e, dst, zrows, zden)
    return scatter


_scatter_a = _make_scatter(0)
_scatter_b = _make_scatter(1)


def _final_body(ha_ref, da_ref, hb_ref, db_ref, x_ref, g_ref, b_ref, out_ref):
    hs = ha_ref[0] + ha_ref[1] + hb_ref[0] + hb_ref[1]
    dn = da_ref[0] + da_ref[1] + db_ref[0] + db_ref[1]
    dn = jnp.where(dn == 0.0, 1.0, dn)
    h = hs / dn[:, None] + x_ref[...]
    mean = jnp.mean(h, axis=1, keepdims=True)
    cen = h - mean
    var = jnp.mean(cen * cen, axis=1, keepdims=True)
    out_ref[...] = cen * lax.rsqrt(var + 1e-6) * g_ref[...] + b_ref[...]


@jax.jit
def _final(ha, da, hb, db, x, gamma, beta):
    grid = pl.cdiv(N, BN)
    hspec = pl.BlockSpec((2, BN, D), lambda i: (0, i, 0))
    dspec = pl.BlockSpec((2, BN), lambda i: (0, i))
    return pl.pallas_call(
        _final_body,
        grid=(grid,),
        in_specs=[
            hspec, dspec, hspec, dspec,
            pl.BlockSpec((BN, D), lambda i: (i, 0)),
            pl.BlockSpec((1, D), lambda i: (0, 0)),
            pl.BlockSpec((1, D), lambda i: (0, 0)),
        ],
        out_specs=pl.BlockSpec((BN, D), lambda i: (i, 0)),
        out_shape=jax.ShapeDtypeStruct((N, D), jnp.float32),
    )(ha, da, hb, db, x, gamma, beta)


def kernel(x, edge_index, edge_h, edge_qrh, W_msg, W_q, W_k, gamma, beta):
    src = edge_index[0].astype(jnp.int32)
    dst = edge_index[1].astype(jnp.int32)
    temp = jnp.float32(D ** 0.5)

    wmxT = W_msg[:, :D].T.astype(jnp.bfloat16)
    wmhT = W_msg[:, D:].T.astype(jnp.bfloat16)
    m = (W_q.T @ W_k) / temp
    mx = m[:, :D].astype(jnp.bfloat16)
    mh = m[:, D:].astype(jnp.bfloat16)

    xs_a = _gather_a(x, src)
    wmsg_a, e2d_a = _edge_tc_a(xs_a, edge_h, edge_qrh, wmxT, wmhT, mx, mh)
    xs_b = _gather_b(x, src)
    wmsg_b, e2d_b = _edge_tc_b(xs_b, edge_h, edge_qrh, wmxT, wmhT, mx, mh)
    ha, da = _scatter_a(wmsg_a, e2d_a.reshape(H), dst)
    hb, db = _scatter_b(wmsg_b, e2d_b.reshape(H), dst)
    return _final(ha, da, hb, db, x, gamma.reshape(1, D), beta.reshape(1, D))


# speedup vs baseline: 1.2185x; 1.1252x over previous
import functools

import jax
import jax.numpy as jnp
from jax import lax
from jax.experimental import pallas as pl
from jax.experimental.pallas import tpu as pltpu
from jax.experimental.pallas import tpu_sc as plsc

N = 10000
E = 320000
D = 128

NSLC = 5
S = E // NSLC
NW = 32
EWS = S // NW
CBS = 80
NCBS = EWS // CBS

BES = 2560
BN = 512
NP = 10240
RPT = NP // 16

_mesh = functools.partial(
    plsc.VectorSubcoreMesh, core_axis_name="c", subcore_axis_name="s")


def _make_gather_body(slc):
    def body(x_hbm, src_hbm, out_hbm,
             idx0, idx1, rows0, rows1,
             i_sem0, i_sem1, g_sem0, g_sem1, w_sem0, w_sem1):
        w = lax.axis_index("c") * 16 + lax.axis_index("s")
        base = slc * S + w * EWS
        obase = w * EWS
        bufs = ((idx0, rows0, i_sem0, g_sem0, w_sem0),
                (idx1, rows1, i_sem1, g_sem1, w_sem1))

        def start_idx(i, b):
            idx_v, _, i_sem, _, _ = bufs[b]
            pltpu.async_copy(src_hbm.at[pl.ds(base + i * CBS, CBS)], idx_v, i_sem)

        def drain_idx(i, b):
            idx_v, _, i_sem, _, _ = bufs[b]
            pltpu.make_async_copy(src_hbm.at[pl.ds(base + i * CBS, CBS)], idx_v,
                                  i_sem).wait()

        def fire_gather(b):
            idx_v, rows_v, _, g_sem, _ = bufs[b]
            pltpu.async_copy(x_hbm.at[idx_v], rows_v, g_sem)

        def drain_gather(b):
            idx_v, rows_v, _, g_sem, _ = bufs[b]
            pltpu.make_async_copy(x_hbm.at[idx_v], rows_v, g_sem).wait()

        def fire_write(i, b):
            _, rows_v, _, _, w_sem = bufs[b]
            pltpu.async_copy(rows_v, out_hbm.at[pl.ds(obase + i * CBS, CBS)], w_sem)

        def drain_write(i, b):
            _, rows_v, _, _, w_sem = bufs[b]
            pltpu.make_async_copy(rows_v, out_hbm.at[pl.ds(obase + i * CBS, CBS)],
                                  w_sem).wait()

        def chunk(i, b):
            drain_idx(i, b)
            pl.when(i >= 2)(lambda: drain_write(i - 2, b))
            fire_gather(b)

            def _prev():
                drain_gather(1 - b)
                fire_write(i - 1, 1 - b)
            pl.when(i >= 1)(_prev)
            pl.when(i + 1 < NCBS)(lambda: start_idx(i + 1, 1 - b))

        start_idx(0, 0)

        def pair(g, carry):
            chunk(2 * g, 0)
            chunk(2 * g + 1, 1)
            return carry

        lax.fori_loop(0, (NCBS - 1) // 2, pair, 0)
        chunk(NCBS - 1, 0)
        drain_gather(0)
        fire_write(NCBS - 1, 0)
        drain_write(NCBS - 2, 1)
        drain_write(NCBS - 1, 0)

    return body


def _make_gather(slc):
    @jax.jit
    def gather(x, src):
        k = pl.kernel(
            _make_gather_body(slc),
            out_type=jax.ShapeDtypeStruct((S, D), jnp.float32),
            mesh=_mesh(),
            scratch_types=[
                pltpu.VMEM((CBS,), jnp.int32),
                pltpu.VMEM((CBS,), jnp.int32),
                pltpu.VMEM((CBS, D), jnp.float32),
                pltpu.VMEM((CBS, D), jnp.float32),
                pltpu.SemaphoreType.DMA,
                pltpu.SemaphoreType.DMA,
                pltpu.SemaphoreType.DMA,
                pltpu.SemaphoreType.DMA,
                pltpu.SemaphoreType.DMA,
                pltpu.SemaphoreType.DMA,
            ],
        )
        return k(x, src)
    return gather


_gathers = [_make_gather(k) for k in range(NSLC)]


def _edge_body(xs_ref, eh_ref, eq_ref, wmx_ref, wmh_ref, mx_ref, mh_ref,
               wmsg_ref, e_ref):
    xs = xs_ref[...]
    eh = eh_ref[...]
    eq = eq_ref[...]
    xsb = xs.astype(jnp.bfloat16)
    ehb = eh.astype(jnp.bfloat16)
    eqb = eq.astype(jnp.bfloat16)
    pre = (jnp.dot(xsb, wmx_ref[...], preferred_element_type=jnp.float32)
           + jnp.dot(ehb, wmh_ref[...], preferred_element_type=jnp.float32))
    msg = jnp.where(pre > 0, pre, 0.01 * pre)
    zx = jnp.dot(eqb, mx_ref[...], preferred_element_type=jnp.float32)
    zh = jnp.dot(eqb, mh_ref[...], preferred_element_type=jnp.float32)
    att = jnp.sum(zx * xs + zh * eh, axis=1)
    e = jnp.exp(att)
    wmsg_ref[...] = e[:, None] * msg
    e_ref[...] = e[None, None, :]


def _make_edge_tc(slc):
    off = slc * (S // BES)

    @jax.jit
    def edge_tc(xs, eh, eq, wmxT, wmhT, mx, mh):
        grid = S // BES
        wspec = pl.BlockSpec((D, D), lambda i: (0, 0))
        return pl.pallas_call(
            _edge_body,
            grid=(grid,),
            in_specs=[
                pl.BlockSpec((BES, D), lambda i: (i, 0)),
                pl.BlockSpec((BES, D), lambda i: (i + off, 0)),
                pl.BlockSpec((BES, D), lambda i: (i + off, 0)),
                wspec, wspec, wspec, wspec,
            ],
            out_specs=[
                pl.BlockSpec((BES, D), lambda i: (i, 0)),
                pl.BlockSpec((1, 1, BES), lambda i: (i, 0, 0)),
            ],
            out_shape=[
                jax.ShapeDtypeStruct((S, D), jnp.float32),
                jax.ShapeDtypeStruct((S // BES, 1, BES), jnp.float32),
            ],
        )(xs, eh, eq, wmxT, wmhT, mx, mh)
    return edge_tc


_edge_tcs = [_make_edge_tc(k) for k in range(NSLC)]


def _make_scatter_body(slcs):
    nsrc = len(slcs)

    def body(*args):
        srcs = [(args[2 * k], args[2 * k + 1]) for k in range(nsrc)]
        (zrows_hbm, zden_hbm, dst_hbm, hagg_hbm, den_hbm,
         rows0, rows1, ev0, ev1, idx0, idx1,
         table, den_sp, in_sem0, in_sem1, sc_sem0, sc_sem1) = args[2 * nsrc:]
        c = lax.axis_index("c")
        s = lax.axis_index("s")
        w = c * 16 + s

        pltpu.sync_copy(zrows_hbm.at[pl.ds(s * RPT, RPT)],
                        table.at[pl.ds(s * RPT, RPT)])
        pltpu.sync_copy(zden_hbm.at[pl.ds(s * RPT, RPT)],
                        den_sp.at[pl.ds(s * RPT, RPT)])
        plsc.subcore_barrier()

        bufs = ((rows0, ev0, idx0, in_sem0, sc_sem0),
                (rows1, ev1, idx1, in_sem1, sc_sem1))

        def run_slice(wmsg_hbm, e_hbm, slc):
            base = w * EWS
            dbase = slc * S + w * EWS

            def start_inputs(i, b):
                rows_v, ev_v, idx_v, in_sem, _ = bufs[b]
                off = base + i * CBS
                pltpu.async_copy(wmsg_hbm.at[pl.ds(off, CBS)], rows_v, in_sem)
                pltpu.async_copy(e_hbm.at[pl.ds(off, CBS)], ev_v, in_sem)
                pltpu.async_copy(dst_hbm.at[pl.ds(dbase + i * CBS, CBS)], idx_v,
                                 in_sem)

            def drain_inputs(i, b):
                rows_v, ev_v, idx_v, in_sem, _ = bufs[b]
                off = base + i * CBS
                pltpu.make_async_copy(wmsg_hbm.at[pl.ds(off, CBS)], rows_v,
                                      in_sem).wait()
                pltpu.make_async_copy(e_hbm.at[pl.ds(off, CBS)], ev_v,
                                      in_sem).wait()
                pltpu.make_async_copy(dst_hbm.at[pl.ds(dbase + i * CBS, CBS)],
                                      idx_v, in_sem).wait()

            def fire_scatters(b):
                rows_v, ev_v, idx_v, _, sc_sem = bufs[b]
                pltpu.async_copy(rows_v, table.at[idx_v], sc_sem, add=True)
                pltpu.async_copy(ev_v, den_sp.at[idx_v], sc_sem, add=True)

            def drain_scatters(b):
                rows_v, ev_v, idx_v, _, sc_sem = bufs[b]
                pltpu.make_async_copy(rows_v, table.at[idx_v], sc_sem).wait()
                pltpu.make_async_copy(ev_v, den_sp.at[idx_v], sc_sem).wait()

            start_inputs(0, 0)

            def pair(g, carry):
                i0 = 2 * g
                drain_inputs(i0, 0)
                fire_scatters(0)
                pl.when(i0 > 0)(lambda: drain_scatters(1))
                start_inputs(i0 + 1, 1)

                drain_inputs(i0 + 1, 1)
                fire_scatters(1)
                drain_scatters(0)
                start_inputs(i0 + 2, 0)
                return carry

            lax.fori_loop(0, (NCBS - 1) // 2, pair, 0)
            drain_inputs(NCBS - 1, 0)
            fire_scatters(0)
            drain_scatters(1)
            drain_scatters(0)

        for k, (wmsg_hbm, e_hbm) in enumerate(srcs):
            run_slice(wmsg_hbm, e_hbm, slcs[k])

        plsc.subcore_barrier()
        pltpu.sync_copy(table.at[pl.ds(s * RPT, RPT)],
                        hagg_hbm.at[c, pl.ds(s * RPT, RPT)])
        pltpu.sync_copy(den_sp.at[pl.ds(s * RPT, RPT)],
                        den_hbm.at[c, pl.ds(s * RPT, RPT)])

    return body


def _make_scatter(slcs):
    nsrc = len(slcs)

    @jax.jit
    def scatter(*args):
        dst = args[-1]
        zrows = jnp.zeros((NP, D), jnp.float32)
        zden = jnp.zeros((NP,), jnp.float32)
        k = pl.kernel(
            _make_scatter_body(slcs),
            out_type=[
                jax.ShapeDtypeStruct((2, NP, D), jnp.float32),
                jax.ShapeDtypeStruct((2, NP), jnp.float32),
            ],
            mesh=_mesh(),
            scratch_types=[
                pltpu.VMEM((CBS, D), jnp.float32),
                pltpu.VMEM((CBS, D), jnp.float32),
                pltpu.VMEM((CBS,), jnp.float32),
                pltpu.VMEM((CBS,), jnp.float32),
                pltpu.VMEM((CBS,), jnp.int32),
                pltpu.VMEM((CBS,), jnp.int32),
                pltpu.VMEM_SHARED((NP, D), jnp.float32),
                pltpu.VMEM_SHARED((NP,), jnp.float32),
                pltpu.SemaphoreType.DMA,
                pltpu.SemaphoreType.DMA,
                pltpu.SemaphoreType.DMA,
                pltpu.SemaphoreType.DMA,
            ],
        )
        return k(*args[:-1], zrows, zden, dst)
    return scatter


_scatter_a = _make_scatter((0, 1, 2))
_scatter_b = _make_scatter((3, 4))


def _final_body(ha_ref, da_ref, hb_ref, db_ref, x_ref, g_ref, b_ref, out_ref):
    hs = ha_ref[0] + ha_ref[1] + hb_ref[0] + hb_ref[1]
    dn = da_ref[0] + da_ref[1] + db_ref[0] + db_ref[1]
    dn = jnp.where(dn == 0.0, 1.0, dn)
    h = hs / dn[:, None] + x_ref[...]
    mean = jnp.mean(h, axis=1, keepdims=True)
    cen = h - mean
    var = jnp.mean(cen * cen, axis=1, keepdims=True)
    out_ref[...] = cen * lax.rsqrt(var + 1e-6) * g_ref[...] + b_ref[...]


@jax.jit
def _final(ha, da, hb, db, x, gamma, beta):
    grid = pl.cdiv(N, BN)
    hspec = pl.BlockSpec((2, BN, D), lambda i: (0, i, 0))
    dspec = pl.BlockSpec((2, BN), lambda i: (0, i))
    return pl.pallas_call(
        _final_body,
        grid=(grid,),
        in_specs=[
            hspec, dspec, hspec, dspec,
            pl.BlockSpec((BN, D), lambda i: (i, 0)),
            pl.BlockSpec((1, D), lambda i: (0, 0)),
            pl.BlockSpec((1, D), lambda i: (0, 0)),
        ],
        out_specs=pl.BlockSpec((BN, D), lambda i: (i, 0)),
        out_shape=jax.ShapeDtypeStruct((N, D), jnp.float32),
    )(ha, da, hb, db, x, gamma, beta)


def kernel(x, edge_index, edge_h, edge_qrh, W_msg, W_q, W_k, gamma, beta):
    src = edge_index[0].astype(jnp.int32)
    dst = edge_index[1].astype(jnp.int32)
    temp = jnp.float32(D ** 0.5)

    wmxT = W_msg[:, :D].T.astype(jnp.bfloat16)
    wmhT = W_msg[:, D:].T.astype(jnp.bfloat16)
    m = (W_q.T @ W_k) / temp
    mx = m[:, :D].astype(jnp.bfloat16)
    mh = m[:, D:].astype(jnp.bfloat16)

    wm, ev = [], []
    for k in range(NSLC):
        xs_k = _gathers[k](x, src)
        wm_k, e2d_k = _edge_tcs[k](xs_k, edge_h, edge_qrh, wmxT, wmhT, mx, mh)
        wm.append(wm_k)
        ev.append(e2d_k.reshape(S))
    ha, da = _scatter_a(wm[0], ev[0], wm[1], ev[1], wm[2], ev[2], dst)
    hb, db = _scatter_b(wm[3], ev[3], wm[4], ev[4], dst)
    return _final(ha, da, hb, db, x, gamma.reshape(1, D), beta.reshape(1, D))


# speedup vs baseline: 1.2303x; 1.0097x over previous
import functools

import jax
import jax.numpy as jnp
from jax import lax
from jax.experimental import pallas as pl
from jax.experimental.pallas import tpu as pltpu
from jax.experimental.pallas import tpu_sc as plsc

N = 10000
E = 320000
D = 128

NSLC = 5
S = E // NSLC
NW = 32
EWS = S // NW
CBS = 80
NCBS = EWS // CBS

BES = 3200
BN = 512
NP = 10240
RPT = NP // 16

_mesh = functools.partial(
    plsc.VectorSubcoreMesh, core_axis_name="c", subcore_axis_name="s")


def _make_gather_body(slc):
    def body(x_hbm, src_hbm, out_hbm,
             idx0, idx1, rows0, rows1,
             i_sem0, i_sem1, g_sem0, g_sem1, w_sem0, w_sem1):
        w = lax.axis_index("c") * 16 + lax.axis_index("s")
        base = slc * S + w * EWS
        obase = w * EWS
        bufs = ((idx0, rows0, i_sem0, g_sem0, w_sem0),
                (idx1, rows1, i_sem1, g_sem1, w_sem1))

        def start_idx(i, b):
            idx_v, _, i_sem, _, _ = bufs[b]
            pltpu.async_copy(src_hbm.at[pl.ds(base + i * CBS, CBS)], idx_v, i_sem)

        def drain_idx(i, b):
            idx_v, _, i_sem, _, _ = bufs[b]
            pltpu.make_async_copy(src_hbm.at[pl.ds(base + i * CBS, CBS)], idx_v,
                                  i_sem).wait()

        def fire_gather(b):
            idx_v, rows_v, _, g_sem, _ = bufs[b]
            pltpu.async_copy(x_hbm.at[idx_v], rows_v, g_sem)

        def drain_gather(b):
            idx_v, rows_v, _, g_sem, _ = bufs[b]
            pltpu.make_async_copy(x_hbm.at[idx_v], rows_v, g_sem).wait()

        def fire_write(i, b):
            _, rows_v, _, _, w_sem = bufs[b]
            pltpu.async_copy(rows_v, out_hbm.at[pl.ds(obase + i * CBS, CBS)], w_sem)

        def drain_write(i, b):
            _, rows_v, _, _, w_sem = bufs[b]
            pltpu.make_async_copy(rows_v, out_hbm.at[pl.ds(obase + i * CBS, CBS)],
                                  w_sem).wait()

        def chunk(i, b):
            drain_idx(i, b)
            pl.when(i >= 2)(lambda: drain_write(i - 2, b))
            fire_gather(b)

            def _prev():
                drain_gather(1 - b)
                fire_write(i - 1, 1 - b)
            pl.when(i >= 1)(_prev)
            pl.when(i + 1 < NCBS)(lambda: start_idx(i + 1, 1 - b))

        start_idx(0, 0)

        def pair(g, carry):
            chunk(2 * g, 0)
            chunk(2 * g + 1, 1)
            return carry

        lax.fori_loop(0, (NCBS - 1) // 2, pair, 0)
        chunk(NCBS - 1, 0)
        drain_gather(0)
        fire_write(NCBS - 1, 0)
        drain_write(NCBS - 2, 1)
        drain_write(NCBS - 1, 0)

    return body


def _make_gather(slc):
    @jax.jit
    def gather(x, src):
        k = pl.kernel(
            _make_gather_body(slc),
            out_type=jax.ShapeDtypeStruct((S, D), jnp.float32),
            mesh=_mesh(),
            scratch_types=[
                pltpu.VMEM((CBS,), jnp.int32),
                pltpu.VMEM((CBS,), jnp.int32),
                pltpu.VMEM((CBS, D), jnp.float32),
                pltpu.VMEM((CBS, D), jnp.float32),
                pltpu.SemaphoreType.DMA,
                pltpu.SemaphoreType.DMA,
                pltpu.SemaphoreType.DMA,
                pltpu.SemaphoreType.DMA,
                pltpu.SemaphoreType.DMA,
                pltpu.SemaphoreType.DMA,
            ],
        )
        return k(x, src)
    return gather


_gathers = [_make_gather(k) for k in range(NSLC)]


def _edge_body(xs_ref, eh_ref, eq_ref, wmx_ref, wmh_ref, mx_ref, mh_ref,
               wmsg_ref, e_ref):
    xs = xs_ref[...]
    eh = eh_ref[...]
    eq = eq_ref[...]
    xsb = xs.astype(jnp.bfloat16)
    ehb = eh.astype(jnp.bfloat16)
    eqb = eq.astype(jnp.bfloat16)
    pre = (jnp.dot(xsb, wmx_ref[...], preferred_element_type=jnp.float32)
           + jnp.dot(ehb, wmh_ref[...], preferred_element_type=jnp.float32))
    msg = jnp.where(pre > 0, pre, 0.01 * pre)
    zx = jnp.dot(eqb, mx_ref[...], preferred_element_type=jnp.float32)
    zh = jnp.dot(eqb, mh_ref[...], preferred_element_type=jnp.float32)
    att = jnp.sum(zx * xs + zh * eh, axis=1)
    e = jnp.exp(att)
    wmsg_ref[...] = e[:, None] * msg
    e_ref[...] = e[None, None, :]


def _make_edge_tc(slc):
    off = slc * (S // BES)

    @jax.jit
    def edge_tc(xs, eh, eq, wmxT, wmhT, mx, mh):
        grid = S // BES
        wspec = pl.BlockSpec((D, D), lambda i: (0, 0))
        return pl.pallas_call(
            _edge_body,
            grid=(grid,),
            in_specs=[
                pl.BlockSpec((BES, D), lambda i: (i, 0)),
                pl.BlockSpec((BES, D), lambda i: (i + off, 0)),
                pl.BlockSpec((BES, D), lambda i: (i + off, 0)),
                wspec, wspec, wspec, wspec,
            ],
            out_specs=[
                pl.BlockSpec((BES, D), lambda i: (i, 0)),
                pl.BlockSpec((1, 1, BES), lambda i: (i, 0, 0)),
            ],
            out_shape=[
                jax.ShapeDtypeStruct((S, D), jnp.float32),
                jax.ShapeDtypeStruct((S // BES, 1, BES), jnp.float32),
            ],
        )(xs, eh, eq, wmxT, wmhT, mx, mh)
    return edge_tc


_edge_tcs = [_make_edge_tc(k) for k in range(NSLC)]


def _make_scatter_body(slcs):
    nsrc = len(slcs)

    def body(*args):
        srcs = [(args[2 * k], args[2 * k + 1]) for k in range(nsrc)]
        (zrows_hbm, zden_hbm, dst_hbm, hagg_hbm, den_hbm,
         rows0, rows1, ev0, ev1, idx0, idx1,
         table, den_sp, in_sem0, in_sem1, sc_sem0, sc_sem1) = args[2 * nsrc:]
        c = lax.axis_index("c")
        s = lax.axis_index("s")
        w = c * 16 + s

        pltpu.sync_copy(zrows_hbm.at[pl.ds(s * RPT, RPT)],
                        table.at[pl.ds(s * RPT, RPT)])
        pltpu.sync_copy(zden_hbm.at[pl.ds(s * RPT, RPT)],
                        den_sp.at[pl.ds(s * RPT, RPT)])
        plsc.subcore_barrier()

        bufs = ((rows0, ev0, idx0, in_sem0, sc_sem0),
                (rows1, ev1, idx1, in_sem1, sc_sem1))

        def run_slice(wmsg_hbm, e_hbm, slc):
            base = w * EWS
            dbase = slc * S + w * EWS

            def start_inputs(i, b):
                rows_v, ev_v, idx_v, in_sem, _ = bufs[b]
                off = base + i * CBS
                pltpu.async_copy(wmsg_hbm.at[pl.ds(off, CBS)], rows_v, in_sem)
                pltpu.async_copy(e_hbm.at[pl.ds(off, CBS)], ev_v, in_sem)
                pltpu.async_copy(dst_hbm.at[pl.ds(dbase + i * CBS, CBS)], idx_v,
                                 in_sem)

            def drain_inputs(i, b):
                rows_v, ev_v, idx_v, in_sem, _ = bufs[b]
                off = base + i * CBS
                pltpu.make_async_copy(wmsg_hbm.at[pl.ds(off, CBS)], rows_v,
                                      in_sem).wait()
                pltpu.make_async_copy(e_hbm.at[pl.ds(off, CBS)], ev_v,
                                      in_sem).wait()
                pltpu.make_async_copy(dst_hbm.at[pl.ds(dbase + i * CBS, CBS)],
                                      idx_v, in_sem).wait()

            def fire_scatters(b):
                rows_v, ev_v, idx_v, _, sc_sem = bufs[b]
                pltpu.async_copy(rows_v, table.at[idx_v], sc_sem, add=True)
                pltpu.async_copy(ev_v, den_sp.at[idx_v], sc_sem, add=True)

            def drain_scatters(b):
                rows_v, ev_v, idx_v, _, sc_sem = bufs[b]
                pltpu.make_async_copy(rows_v, table.at[idx_v], sc_sem).wait()
                pltpu.make_async_copy(ev_v, den_sp.at[idx_v], sc_sem).wait()

            start_inputs(0, 0)

            def pair(g, carry):
                i0 = 2 * g
                drain_inputs(i0, 0)
                fire_scatters(0)
                pl.when(i0 > 0)(lambda: drain_scatters(1))
                start_inputs(i0 + 1, 1)

                drain_inputs(i0 + 1, 1)
                fire_scatters(1)
                drain_scatters(0)
                start_inputs(i0 + 2, 0)
                return carry

            lax.fori_loop(0, (NCBS - 1) // 2, pair, 0)
            drain_inputs(NCBS - 1, 0)
            fire_scatters(0)
            drain_scatters(1)
            drain_scatters(0)

        for k, (wmsg_hbm, e_hbm) in enumerate(srcs):
            run_slice(wmsg_hbm, e_hbm, slcs[k])

        plsc.subcore_barrier()
        pltpu.sync_copy(table.at[pl.ds(s * RPT, RPT)],
                        hagg_hbm.at[c, pl.ds(s * RPT, RPT)])
        pltpu.sync_copy(den_sp.at[pl.ds(s * RPT, RPT)],
                        den_hbm.at[c, pl.ds(s * RPT, RPT)])

    return body


def _make_scatter(slcs):
    nsrc = len(slcs)

    @jax.jit
    def scatter(*args):
        dst = args[-1]
        zrows = jnp.zeros((NP, D), jnp.float32)
        zden = jnp.zeros((NP,), jnp.float32)
        k = pl.kernel(
            _make_scatter_body(slcs),
            out_type=[
                jax.ShapeDtypeStruct((2, NP, D), jnp.float32),
                jax.ShapeDtypeStruct((2, NP), jnp.float32),
            ],
            mesh=_mesh(),
            scratch_types=[
                pltpu.VMEM((CBS, D), jnp.float32),
                pltpu.VMEM((CBS, D), jnp.float32),
                pltpu.VMEM((CBS,), jnp.float32),
                pltpu.VMEM((CBS,), jnp.float32),
                pltpu.VMEM((CBS,), jnp.int32),
                pltpu.VMEM((CBS,), jnp.int32),
                pltpu.VMEM_SHARED((NP, D), jnp.float32),
                pltpu.VMEM_SHARED((NP,), jnp.float32),
                pltpu.SemaphoreType.DMA,
                pltpu.SemaphoreType.DMA,
                pltpu.SemaphoreType.DMA,
                pltpu.SemaphoreType.DMA,
            ],
        )
        return k(*args[:-1], zrows, zden, dst)
    return scatter


_scatter_a = _make_scatter((0, 1, 2))
_scatter_b = _make_scatter((3, 4))


def _final_body(ha_ref, da_ref, hb_ref, db_ref, x_ref, g_ref, b_ref, out_ref):
    hs = ha_ref[0] + ha_ref[1] + hb_ref[0] + hb_ref[1]
    dn = da_ref[0] + da_ref[1] + db_ref[0] + db_ref[1]
    dn = jnp.where(dn == 0.0, 1.0, dn)
    h = hs / dn[:, None] + x_ref[...]
    mean = jnp.mean(h, axis=1, keepdims=True)
    cen = h - mean
    var = jnp.mean(cen * cen, axis=1, keepdims=True)
    out_ref[...] = cen * lax.rsqrt(var + 1e-6) * g_ref[...] + b_ref[...]


@jax.jit
def _final(ha, da, hb, db, x, gamma, beta):
    grid = pl.cdiv(N, BN)
    hspec = pl.BlockSpec((2, BN, D), lambda i: (0, i, 0))
    dspec = pl.BlockSpec((2, BN), lambda i: (0, i))
    return pl.pallas_call(
        _final_body,
        grid=(grid,),
        in_specs=[
            hspec, dspec, hspec, dspec,
            pl.BlockSpec((BN, D), lambda i: (i, 0)),
            pl.BlockSpec((1, D), lambda i: (0, 0)),
            pl.BlockSpec((1, D), lambda i: (0, 0)),
        ],
        out_specs=pl.BlockSpec((BN, D), lambda i: (i, 0)),
        out_shape=jax.ShapeDtypeStruct((N, D), jnp.float32),
    )(ha, da, hb, db, x, gamma, beta)


def kernel(x, edge_index, edge_h, edge_qrh, W_msg, W_q, W_k, gamma, beta):
    src = edge_index[0].astype(jnp.int32)
    dst = edge_index[1].astype(jnp.int32)
    temp = jnp.float32(D ** 0.5)

    wmxT = W_msg[:, :D].T.astype(jnp.bfloat16)
    wmhT = W_msg[:, D:].T.astype(jnp.bfloat16)
    m = (W_q.T @ W_k) / temp
    mx = m[:, :D].astype(jnp.bfloat16)
    mh = m[:, D:].astype(jnp.bfloat16)

    wm, ev = [], []
    for k in range(NSLC):
        xs_k = _gathers[k](x, src)
        wm_k, e2d_k = _edge_tcs[k](xs_k, edge_h, edge_qrh, wmxT, wmhT, mx, mh)
        wm.append(wm_k)
        ev.append(e2d_k.reshape(S))
    ha, da = _scatter_a(wm[0], ev[0], wm[1], ev[1], wm[2], ev[2], dst)
    hb, db = _scatter_b(wm[3], ev[3], wm[4], ev[4], dst)
    return _final(ha, da, hb, db, x, gamma.reshape(1, D), beta.reshape(1, D))
